# Initial kernel scaffold; baseline (speedup 1.0000x reference)
import jax
import jax.numpy as jnp
from jax.experimental import pallas as pl


def kernel(x, edge_index, weight, bias):
    raise NotImplementedError("write your pallas kernel here")


# speedup vs baseline: 6.4976x; 6.4976x over previous
import functools

import jax
import jax.numpy as jnp
from jax import lax
from jax.experimental import pallas as pl
from jax.experimental.pallas import tpu as pltpu
from jax.experimental.pallas import tpu_sc as plsc

N = 10000
E = 320000
D = 128
NC, NS, L = 2, 16, 16
NPAD = 10240
CH = 128
NCHUNKS = E // CH
SL = NPAD // NS
RPS = NPAD // NS
ORPS = N // NS


def _rsqrt_newton(d):
    yi = jnp.int32(0x5F3759DF) - lax.shift_right_logical(
        lax.bitcast_convert_type(d, jnp.int32), 1)
    y = lax.bitcast_convert_type(yi, jnp.float32)
    for _ in range(3):
        y = y * (1.5 - 0.5 * d * y * y)
    return y


def _deg_body(eidx_hbm, norm_hbm, idx_v, ones_v, slice_v, deg_sh):
    c = lax.axis_index("c")
    s = lax.axis_index("s")
    for j in range(CH // L):
        ones_v[pl.ds(j * L, L)] = jnp.ones((L,), jnp.float32)
    for j in range(SL // L):
        slice_v[pl.ds(j * L, L)] = jnp.zeros((L,), jnp.float32)
    pltpu.sync_copy(slice_v, deg_sh.at[pl.ds(s * SL, SL)])
    plsc.subcore_barrier()

    ni = (NCHUNKS + NS - 1) // NS

    def body(i, carry):
        chunk = i * NS + s

        @pl.when(chunk < NCHUNKS)
        def _():
            base = c * E + chunk * CH
            pltpu.sync_copy(eidx_hbm.at[pl.ds(base, CH)], idx_v)
            pltpu.sync_copy(ones_v, deg_sh.at[idx_v], add=True)

        return carry

    lax.fori_loop(0, ni, body, 0)
    plsc.subcore_barrier()

    pltpu.sync_copy(deg_sh.at[pl.ds(s * SL, SL)], slice_v)
    for j in range(SL // L):
        d = jnp.maximum(slice_v[pl.ds(j * L, L)], 1.0)
        slice_v[pl.ds(j * L, L)] = _rsqrt_newton(d)
    pltpu.sync_copy(slice_v, norm_hbm.at[pl.ds(c * NPAD + s * SL, SL)])


_deg_call = pl.kernel(
    _deg_body,
    out_type=jax.ShapeDtypeStruct((NC * NPAD,), jnp.float32),
    mesh=plsc.VectorSubcoreMesh(core_axis_name="c", subcore_axis_name="s"),
    scratch_types=[
        pltpu.VMEM((CH,), jnp.int32),
        pltpu.VMEM((CH,), jnp.float32),
        pltpu.VMEM((SL,), jnp.float32),
        pltpu.VMEM_SHARED((NPAD,), jnp.float32),
    ],
)


def _agg_body(feat_hbm, src_hbm, dst_hbm, part_hbm,
              src_v, dst_v, rows_v, acc_sh, sem):
    c = lax.axis_index("c")
    s = lax.axis_index("s")

    def zrow(i, carry):
        for j in range(D // L):
            rows_v[i, pl.ds(j * L, L)] = jnp.zeros((L,), jnp.float32)
        return carry

    lax.fori_loop(0, CH, zrow, 0)
    for k in range(RPS // CH):
        pltpu.sync_copy(rows_v, acc_sh.at[pl.ds(s * RPS + k * CH, CH)])
    plsc.subcore_barrier()

    half = NCHUNKS // NC
    ni = (half + NS - 1) // NS

    def body(i, carry):
        local = i * NS + s

        @pl.when(local < half)
        def _():
            base = (c * half + local) * CH
            pltpu.sync_copy(src_hbm.at[pl.ds(base, CH)], src_v)
            pltpu.sync_copy(dst_hbm.at[pl.ds(base, CH)], dst_v)
            pltpu.async_copy(feat_hbm.at[src_v], rows_v, sem).wait()
            pltpu.sync_copy(rows_v, acc_sh.at[dst_v], add=True)

        return carry

    lax.fori_loop(0, ni, body, 0)
    plsc.subcore_barrier()

    for k in range(5):
        r0 = s * RPS + k * CH
        pltpu.sync_copy(acc_sh.at[pl.ds(r0, CH)], rows_v)
        pltpu.sync_copy(rows_v, part_hbm.at[c, pl.ds(r0, CH)])


_agg_call = pl.kernel(
    _agg_body,
    out_type=jax.ShapeDtypeStruct((NC, NPAD, D), jnp.float32),
    mesh=plsc.VectorSubcoreMesh(core_axis_name="c", subcore_axis_name="s"),
    scratch_types=[
        pltpu.VMEM((CH,), jnp.int32),
        pltpu.VMEM((CH,), jnp.int32),
        pltpu.VMEM((CH, D), jnp.float32),
        pltpu.VMEM_SHARED((NPAD, D), jnp.float32),
        pltpu.SemaphoreType.DMA,
    ],
)


_RB = 2000


def _prep_body(x_ref, n_ref, o_ref):
    o_ref[...] = x_ref[...] * n_ref[...]


_prep_call = pl.pallas_call(
    _prep_body,
    grid=(N // _RB,),
    in_specs=[
        pl.BlockSpec((_RB, D), lambda i: (i, 0)),
        pl.BlockSpec((_RB, 1), lambda i: (i, 0)),
    ],
    out_specs=pl.BlockSpec((_RB, D), lambda i: (i, 0)),
    out_shape=jax.ShapeDtypeStruct((N, D), jnp.float32),
)


def _final_body(p_ref, n_ref, w_ref, b_ref, o_ref):
    acc = (p_ref[0] + p_ref[1]) * n_ref[...]
    o_ref[...] = jnp.dot(acc, w_ref[...],
                         preferred_element_type=jnp.float32) + b_ref[...]


_final_call = pl.pallas_call(
    _final_body,
    grid=(N // _RB,),
    in_specs=[
        pl.BlockSpec((NC, _RB, D), lambda i: (0, i, 0)),
        pl.BlockSpec((_RB, 1), lambda i: (i, 0)),
        pl.BlockSpec((D, D), lambda i: (0, 0)),
        pl.BlockSpec((1, D), lambda i: (0, 0)),
    ],
    out_specs=pl.BlockSpec((_RB, D), lambda i: (i, 0)),
    out_shape=jax.ShapeDtypeStruct((N, D), jnp.float32),
)


def kernel(x, edge_index, weight, bias):
    eflat = edge_index.reshape(NC * E)
    norms = _deg_call(eflat)
    n_src = norms[0:N].reshape(N, 1)
    n_dst = norms[NPAD:NPAD + N].reshape(N, 1)
    feat = _prep_call(x, n_src)
    parts = _agg_call(feat, edge_index[0], edge_index[1])
    return _final_call(parts, n_dst, weight, bias.reshape(1, D))


# speedup vs baseline: 14.3837x; 2.2137x over previous
import functools

import jax
import jax.numpy as jnp
from jax import lax
from jax.experimental import pallas as pl
from jax.experimental.pallas import tpu as pltpu
from jax.experimental.pallas import tpu_sc as plsc

N = 10000
E = 320000
D = 128
NC, NS, L = 2, 16, 16
NW = NC * NS
NPAD = 10240
CH = 128
EPAD = NW * 80 * CH
NROWS = EPAD // CH
NREAL = E // CH
SL = NPAD // NS
RPS = NPAD // NS
SHIFT = 14
MASK = (1 << SHIFT) - 1


def _rsqrt_newton(d):
    yi = jnp.int32(0x5F3759DF) - lax.shift_right_logical(
        lax.bitcast_convert_type(d, jnp.int32), 1)
    y = lax.bitcast_convert_type(yi, jnp.float32)
    for _ in range(3):
        y = y * (1.5 - 0.5 * d * y * y)
    return y


_DROWS = NROWS // NS
_DFIRE = 8


def _deg_body(epk_hbm, norm_hbm, epk_v, idx_v, ones_v, slice_v, deg_sh, sem):
    c = lax.axis_index("c")
    s = lax.axis_index("s")
    for j in range(CH // L):
        ones_v[pl.ds(j * L, L)] = jnp.ones((L,), jnp.float32)
    for j in range(SL // L):
        slice_v[pl.ds(j * L, L)] = jnp.zeros((L,), jnp.float32)
    pltpu.sync_copy(slice_v, deg_sh.at[pl.ds(s * SL, SL)])
    pltpu.sync_copy(epk_hbm.at[pl.ds(s * _DROWS, _DROWS)], epk_v)
    sh = SHIFT * (1 - c)

    def unpack(i, carry):
        for j in range(CH // L):
            e = epk_v[i, pl.ds(j * L, L)]
            idx_v[i, pl.ds(j * L, L)] = (
                lax.shift_right_logical(e, sh) & MASK)
        return carry

    lax.fori_loop(0, _DROWS, unpack, 0)
    plsc.subcore_barrier()

    def body(i, carry):
        r0 = i * _DFIRE
        for j in range(_DFIRE):
            @pl.when(s * _DROWS + r0 + j < NREAL)
            def _():
                pltpu.async_copy(ones_v, deg_sh.at[idx_v.at[r0 + j]], sem,
                                 add=True)
        for j in range(_DFIRE):
            @pl.when(s * _DROWS + r0 + j < NREAL)
            def _():
                pltpu.make_async_copy(
                    ones_v, deg_sh.at[idx_v.at[r0 + j]], sem).wait()
        return carry

    lax.fori_loop(0, _DROWS // _DFIRE, body, 0)
    plsc.subcore_barrier()

    pltpu.sync_copy(deg_sh.at[pl.ds(s * SL, SL)], slice_v)
    for j in range(SL // L):
        d = jnp.maximum(slice_v[pl.ds(j * L, L)], 1.0)
        slice_v[pl.ds(j * L, L)] = _rsqrt_newton(d)
    pltpu.sync_copy(slice_v, norm_hbm.at[pl.ds(c * NPAD + s * SL, SL)])


_deg_call = pl.kernel(
    _deg_body,
    out_type=jax.ShapeDtypeStruct((NC * NPAD,), jnp.float32),
    mesh=plsc.VectorSubcoreMesh(core_axis_name="c", subcore_axis_name="s"),
    scratch_types=[
        pltpu.VMEM((_DROWS, CH), jnp.int32),
        pltpu.VMEM((_DROWS, CH), jnp.int32),
        pltpu.VMEM((CH,), jnp.float32),
        pltpu.VMEM((SL,), jnp.float32),
        pltpu.VMEM_SHARED((NPAD,), jnp.float32),
        pltpu.SemaphoreType.DMA,
    ],
)


_AROWS = NROWS // NW


def _agg_body(feat_hbm, epk_hbm, part_hbm,
              epk_v, si_a, si_b, di_v, buf_a, buf_b, acc_sh, sem_a, sem_b):
    c = lax.axis_index("c")
    s = lax.axis_index("s")
    w = c * NS + s

    def zrow(i, carry):
        for j in range(D // L):
            buf_a[i, pl.ds(j * L, L)] = jnp.zeros((L,), jnp.float32)
        return carry

    lax.fori_loop(0, CH, zrow, 0)
    for k in range(RPS // CH):
        pltpu.sync_copy(buf_a, acc_sh.at[pl.ds(s * RPS + k * CH, CH)])
    pltpu.sync_copy(epk_hbm.at[pl.ds(w * _AROWS, _AROWS)], epk_v)

    def unpack_src(row, out_ref):
        for j in range(CH // L):
            e = epk_v[row, pl.ds(j * L, L)]
            out_ref[pl.ds(j * L, L)] = lax.shift_right_logical(e, SHIFT)

    def unpack_dst(row):
        for j in range(CH // L):
            e = epk_v[row, pl.ds(j * L, L)]
            di_v[pl.ds(j * L, L)] = e & MASK

    plsc.subcore_barrier()

    unpack_src(0, si_a)
    pltpu.async_copy(feat_hbm.at[si_a], buf_a, sem_a)

    def body(i2, carry):
        j0 = 2 * i2
        unpack_src(j0 + 1, si_b)
        pltpu.async_copy(feat_hbm.at[si_b], buf_b, sem_b)
        pltpu.make_async_copy(feat_hbm.at[si_a], buf_a, sem_a).wait()
        unpack_dst(j0)
        pltpu.sync_copy(buf_a, acc_sh.at[di_v], add=True)

        @pl.when(j0 + 2 < _AROWS)
        def _():
            unpack_src(j0 + 2, si_a)
            pltpu.async_copy(feat_hbm.at[si_a], buf_a, sem_a)

        pltpu.make_async_copy(feat_hbm.at[si_b], buf_b, sem_b).wait()
        unpack_dst(j0 + 1)
        pltpu.sync_copy(buf_b, acc_sh.at[di_v], add=True)
        return carry

    lax.fori_loop(0, _AROWS // 2, body, 0)
    plsc.subcore_barrier()

    for k in range(RPS // CH):
        r0 = s * RPS + k * CH
        pltpu.sync_copy(acc_sh.at[pl.ds(r0, CH)], buf_a)
        pltpu.sync_copy(buf_a, part_hbm.at[c, pl.ds(r0, CH)])


_agg_call = pl.kernel(
    _agg_body,
    out_type=jax.ShapeDtypeStruct((NC, NPAD, D), jnp.float32),
    mesh=plsc.VectorSubcoreMesh(core_axis_name="c", subcore_axis_name="s"),
    scratch_types=[
        pltpu.VMEM((_AROWS, CH), jnp.int32),
        pltpu.VMEM((CH,), jnp.int32),
        pltpu.VMEM((CH,), jnp.int32),
        pltpu.VMEM((CH,), jnp.int32),
        pltpu.VMEM((CH, D), jnp.float32),
        pltpu.VMEM((CH, D), jnp.float32),
        pltpu.VMEM_SHARED((NPAD, D), jnp.float32),
        pltpu.SemaphoreType.DMA,
        pltpu.SemaphoreType.DMA,
    ],
)


_RB = 2000


def _prep_body(x_ref, n_ref, o_ref):
    o_ref[...] = x_ref[...] * n_ref[...]


_prep_call = pl.pallas_call(
    _prep_body,
    grid=(N // _RB,),
    in_specs=[
        pl.BlockSpec((_RB, D), lambda i: (i, 0)),
        pl.BlockSpec((_RB, 1), lambda i: (i, 0)),
    ],
    out_specs=pl.BlockSpec((_RB, D), lambda i: (i, 0)),
    out_shape=jax.ShapeDtypeStruct((N, D), jnp.float32),
)


def _final_body(p_ref, n_ref, w_ref, b_ref, o_ref):
    acc = (p_ref[0] + p_ref[1]) * n_ref[...]
    o_ref[...] = jnp.dot(acc, w_ref[...],
                         preferred_element_type=jnp.float32) + b_ref[...]


_final_call = pl.pallas_call(
    _final_body,
    grid=(N // _RB,),
    in_specs=[
        pl.BlockSpec((NC, _RB, D), lambda i: (0, i, 0)),
        pl.BlockSpec((_RB, 1), lambda i: (i, 0)),
        pl.BlockSpec((D, D), lambda i: (0, 0)),
        pl.BlockSpec((1, D), lambda i: (0, 0)),
    ],
    out_specs=pl.BlockSpec((_RB, D), lambda i: (i, 0)),
    out_shape=jax.ShapeDtypeStruct((N, D), jnp.float32),
)


def kernel(x, edge_index, weight, bias):
    npad = EPAD - E
    pad_src = jnp.arange(npad, dtype=jnp.int32) % N
    pad_dst = N + jnp.arange(npad, dtype=jnp.int32) % (NPAD - N)
    srcp = jnp.concatenate([edge_index[0], pad_src])
    dstp = jnp.concatenate([edge_index[1], pad_dst])
    epk = ((srcp << SHIFT) | dstp).reshape(NROWS, CH)
    norms = _deg_call(epk)
    n_src = norms[0:N].reshape(N, 1)
    n_dst = norms[NPAD:NPAD + N].reshape(N, 1)
    feat = _prep_call(x, n_src)
    parts = _agg_call(feat, epk)
    return _final_call(parts, n_dst, weight, bias.reshape(1, D))


# speedup vs baseline: 15.8276x; 1.1004x over previous
import jax
import jax.numpy as jnp
from jax import lax
from jax.experimental import pallas as pl
from jax.experimental.pallas import tpu as pltpu
from jax.experimental.pallas import tpu_sc as plsc

N = 10000
E = 320000
D = 128
NC, NS, L = 2, 16, 16
NW = NC * NS
NPAD = 10240
CH = 128
EPAD = NW * 80 * CH
NROWS = EPAD // CH
NREAL = E // CH
SL = NPAD // NS
RPS = NPAD // NS
SHIFT = 14
MASK = (1 << SHIFT) - 1


def _rsqrt_newton(d):
    yi = jnp.int32(0x5F3759DF) - lax.shift_right_logical(
        lax.bitcast_convert_type(d, jnp.int32), 1)
    y = lax.bitcast_convert_type(yi, jnp.float32)
    for _ in range(3):
        y = y * (1.5 - 0.5 * d * y * y)
    return y


_DROWS = NROWS // NS
_DFIRE = 8


def _deg_body(epk_hbm, norm_hbm, epk_v, idx_v, ones_v, slice_v, deg_sh, sem):
    c = lax.axis_index("c")
    s = lax.axis_index("s")
    for j in range(CH // L):
        ones_v[pl.ds(j * L, L)] = jnp.ones((L,), jnp.float32)
    for j in range(SL // L):
        slice_v[pl.ds(j * L, L)] = jnp.zeros((L,), jnp.float32)
    pltpu.sync_copy(slice_v, deg_sh.at[pl.ds(s * SL, SL)])
    pltpu.sync_copy(epk_hbm.at[pl.ds(s * _DROWS, _DROWS)], epk_v)
    sh = SHIFT * (1 - c)

    def unpack(i, carry):
        for j in range(CH // L):
            e = epk_v[i, pl.ds(j * L, L)]
            idx_v[i, pl.ds(j * L, L)] = (
                lax.shift_right_logical(e, sh) & MASK)
        return carry

    lax.fori_loop(0, _DROWS, unpack, 0)
    plsc.subcore_barrier()

    def body(i, carry):
        r0 = i * _DFIRE
        for j in range(_DFIRE):
            @pl.when(s * _DROWS + r0 + j < NREAL)
            def _():
                pltpu.async_copy(ones_v, deg_sh.at[idx_v.at[r0 + j]], sem,
                                 add=True)
        for j in range(_DFIRE):
            @pl.when(s * _DROWS + r0 + j < NREAL)
            def _():
                pltpu.make_async_copy(
                    ones_v, deg_sh.at[idx_v.at[r0 + j]], sem).wait()
        return carry

    lax.fori_loop(0, _DROWS // _DFIRE, body, 0)
    plsc.subcore_barrier()

    pltpu.sync_copy(deg_sh.at[pl.ds(s * SL, SL)], slice_v)

    def rs(i, carry):
        d = jnp.maximum(slice_v[pl.ds(i * L, L)], 1.0)
        slice_v[pl.ds(i * L, L)] = _rsqrt_newton(d)
        return carry

    lax.fori_loop(0, SL // L, rs, 0)
    pltpu.sync_copy(slice_v, norm_hbm.at[pl.ds(c * NPAD + s * SL, SL)])


_deg_call = pl.kernel(
    _deg_body,
    out_type=jax.ShapeDtypeStruct((NC * NPAD,), jnp.float32),
    mesh=plsc.VectorSubcoreMesh(core_axis_name="c", subcore_axis_name="s"),
    scratch_types=[
        pltpu.VMEM((_DROWS, CH), jnp.int32),
        pltpu.VMEM((_DROWS, CH), jnp.int32),
        pltpu.VMEM((CH,), jnp.float32),
        pltpu.VMEM((SL,), jnp.float32),
        pltpu.VMEM_SHARED((NPAD,), jnp.float32),
        pltpu.SemaphoreType.DMA,
    ],
)


_AROWS = NROWS // NW
GCH = 64
_NCHK = _AROWS * CH // GCH
_NBUF = 4


def _agg_body(feat_hbm, epk_hbm, part_hbm,
              epk_v, si_v, di_v, bufs, acc_sh, sems):
    c = lax.axis_index("c")
    s = lax.axis_index("s")
    w = c * NS + s

    def zrow(i, carry):
        for j in range(D // L):
            bufs[0, i, pl.ds(j * L, L)] = jnp.zeros((L,), jnp.float32)
        return carry

    lax.fori_loop(0, GCH, zrow, 0)
    for k in range(RPS // GCH):
        pltpu.sync_copy(bufs.at[0], acc_sh.at[pl.ds(s * RPS + k * GCH, GCH)])
    pltpu.sync_copy(epk_hbm.at[pl.ds(w * _AROWS, _AROWS)], epk_v)

    def unpack_src(row, off, slot):
        for j in range(GCH // L):
            e = epk_v[row, pl.ds(off + j * L, L)]
            si_v[slot, pl.ds(j * L, L)] = lax.shift_right_logical(e, SHIFT)

    def unpack_dst(row, off):
        for j in range(GCH // L):
            e = epk_v[row, pl.ds(off + j * L, L)]
            di_v[pl.ds(j * L, L)] = e & MASK

    def fire(slot):
        pltpu.async_copy(feat_hbm.at[si_v.at[slot]], bufs.at[slot],
                         sems.at[slot])

    def wait(slot):
        pltpu.make_async_copy(feat_hbm.at[si_v.at[slot]], bufs.at[slot],
                              sems.at[slot]).wait()

    plsc.subcore_barrier()

    for k in range(_NBUF - 1):
        unpack_src(k // 2, (k % 2) * GCH, k)
        fire(k)

    def body(i4, carry):
        for k in range(_NBUF):
            wait(k)

            @pl.when(4 * i4 + k + 3 < _NCHK)
            def _():
                k3 = k + 3
                unpack_src(2 * i4 + k3 // 2, (k3 % 2) * GCH, (k + 3) % _NBUF)
                fire((k + 3) % _NBUF)

            unpack_dst(2 * i4 + k // 2, (k % 2) * GCH)
            pltpu.sync_copy(bufs.at[k], acc_sh.at[di_v], add=True)
        return carry

    lax.fori_loop(0, _NCHK // _NBUF, body, 0)
    plsc.subcore_barrier()

    nw_ = RPS // GCH
    pltpu.sync_copy(acc_sh.at[pl.ds(s * RPS, GCH)], bufs.at[0])
    for k in range(nw_):
        r0 = s * RPS + k * GCH
        wd = pltpu.async_copy(bufs.at[k % 2], part_hbm.at[c, pl.ds(r0, GCH)],
                              sems.at[_NBUF])
        if k + 1 < nw_:
            pltpu.sync_copy(acc_sh.at[pl.ds(r0 + GCH, GCH)],
                            bufs.at[(k + 1) % 2])
        wd.wait()


_agg_call = pl.kernel(
    _agg_body,
    out_type=jax.ShapeDtypeStruct((NC, NPAD, D), jnp.float32),
    mesh=plsc.VectorSubcoreMesh(core_axis_name="c", subcore_axis_name="s"),
    scratch_types=[
        pltpu.VMEM((_AROWS, CH), jnp.int32),
        pltpu.VMEM((_NBUF, GCH), jnp.int32),
        pltpu.VMEM((GCH,), jnp.int32),
        pltpu.VMEM((_NBUF, GCH, D), jnp.float32),
        pltpu.VMEM_SHARED((NPAD, D), jnp.float32),
        pltpu.SemaphoreType.DMA((_NBUF + 1,)),
    ],
)


_RB = 2000


def _prep_body(x_ref, n_ref, o_ref):
    o_ref[...] = x_ref[...] * n_ref[...]


_prep_call = pl.pallas_call(
    _prep_body,
    grid=(N // _RB,),
    in_specs=[
        pl.BlockSpec((_RB, D), lambda i: (i, 0)),
        pl.BlockSpec((_RB, 1), lambda i: (i, 0)),
    ],
    out_specs=pl.BlockSpec((_RB, D), lambda i: (i, 0)),
    out_shape=jax.ShapeDtypeStruct((N, D), jnp.float32),
)


def _final_body(p_ref, n_ref, w_ref, b_ref, o_ref):
    acc = (p_ref[0] + p_ref[1]) * n_ref[...]
    o_ref[...] = jnp.dot(acc, w_ref[...],
                         preferred_element_type=jnp.float32) + b_ref[...]


_final_call = pl.pallas_call(
    _final_body,
    grid=(N // _RB,),
    in_specs=[
        pl.BlockSpec((NC, _RB, D), lambda i: (0, i, 0)),
        pl.BlockSpec((_RB, 1), lambda i: (i, 0)),
        pl.BlockSpec((D, D), lambda i: (0, 0)),
        pl.BlockSpec((1, D), lambda i: (0, 0)),
    ],
    out_specs=pl.BlockSpec((_RB, D), lambda i: (i, 0)),
    out_shape=jax.ShapeDtypeStruct((N, D), jnp.float32),
)


def kernel(x, edge_index, weight, bias):
    npad = EPAD - E
    pad_src = jnp.arange(npad, dtype=jnp.int32) % N
    pad_dst = N + jnp.arange(npad, dtype=jnp.int32) % (NPAD - N)
    srcp = jnp.concatenate([edge_index[0], pad_src])
    dstp = jnp.concatenate([edge_index[1], pad_dst])
    epk = ((srcp << SHIFT) | dstp).reshape(NROWS, CH)
    norms = _deg_call(epk)
    n_src = norms[0:N].reshape(N, 1)
    n_dst = norms[NPAD:NPAD + N].reshape(N, 1)
    feat = _prep_call(x, n_src)
    parts = _agg_call(feat, epk)
    return _final_call(parts, n_dst, weight, bias.reshape(1, D))
